# baseline (speedup 1.0000x reference)
import jax
import jax.numpy as jnp
from jax.experimental import pallas as pl


def _bcast_kernel(w_ref, out_ref):
    out_ref[0, :, :] = w_ref[...]


def kernel(tokens, W_pos):
    batch, seq_len = tokens.shape
    d = W_pos.shape[1]
    bs = 2048
    grid = (seq_len // bs, batch)
    out = pl.pallas_call(
        _bcast_kernel,
        grid=grid,
        in_specs=[pl.BlockSpec((bs, d), lambda i, b: (i, 0))],
        out_specs=pl.BlockSpec((1, bs, d), lambda i, b: (b, i, 0)),
        out_shape=jax.ShapeDtypeStruct((batch, seq_len, d), W_pos.dtype),
    )(W_pos[:seq_len])
    return out


# speedup vs baseline: 1.1376x; 1.1376x over previous
import jax
import jax.numpy as jnp
from jax.experimental import pallas as pl


def _bcast_kernel(w_ref, out_ref):
    out_ref[...] = jnp.broadcast_to(w_ref[...][None, :, :], out_ref.shape)


def kernel(tokens, W_pos):
    batch, seq_len = tokens.shape
    d = W_pos.shape[1]
    bs = 1024
    grid = (seq_len // bs,)
    out = pl.pallas_call(
        _bcast_kernel,
        grid=grid,
        in_specs=[pl.BlockSpec((bs, d), lambda i: (i, 0))],
        out_specs=pl.BlockSpec((batch, bs, d), lambda i: (0, i, 0)),
        out_shape=jax.ShapeDtypeStruct((batch, seq_len, d), W_pos.dtype),
    )(W_pos[:seq_len])
    return out
